# baseline (speedup 1.0000x reference)
import jax
import jax.numpy as jnp
from jax.experimental import pallas as pl
from jax.experimental.pallas import tpu as pltpu

H = 1024
FF = 2048
E = 8
TOP_K = 2
T = 512

KF = 4
FC = FF // KF
R = T * TOP_K
MB = 128
JMAX = R // MB

_NEG = -3.0e38


def _router_shared_kernel(x_ref, rw_ref, wg_ref, wu_ref, wd_ref,
                          shared_ref, ts_ref, es_ref, ws_ref,
                          start8_ref, nblk_ref):
    kf = pl.program_id(0)
    x = x_ref[...]

    @pl.when(kf == 0)
    def _router_and_dispatch():
        logits = jax.lax.dot_general(
            x, rw_ref[...], (((1,), (1,)), ((), ())),
            preferred_element_type=jnp.float32)
        ii = jax.lax.broadcasted_iota(jnp.int32, (T, E), 1)
        m1 = jnp.max(logits, axis=1, keepdims=True)
        i1 = jnp.min(jnp.where(logits == m1, ii, E), axis=1, keepdims=True)
        l2 = jnp.where(ii == i1, _NEG, logits)
        m2 = jnp.max(l2, axis=1, keepdims=True)
        i2 = jnp.min(jnp.where(l2 == m2, ii, E), axis=1, keepdims=True)
        w1 = jax.nn.sigmoid(m1 - m2)
        w2 = jax.nn.sigmoid(m2 - m1)

        e_flat = jnp.concatenate([i1, i2], axis=0)
        w_flat = jnp.concatenate([w1, w2], axis=0)
        t_col = jax.lax.broadcasted_iota(jnp.int32, (T, 1), 0)
        t_flat = jnp.concatenate([t_col, t_col], axis=0).astype(jnp.float32)

        onehot = (e_flat == jax.lax.broadcasted_iota(
            jnp.int32, (R, E), 1)).astype(jnp.float32)
        counts = jnp.sum(onehot, axis=0, keepdims=True)
        ue = jax.lax.broadcasted_iota(jnp.int32, (E, E), 0)
        ve = jax.lax.broadcasted_iota(jnp.int32, (E, E), 1)
        ustrict = (ue < ve).astype(jnp.float32)
        offsets = jnp.dot(counts, ustrict, precision=jax.lax.Precision.HIGHEST,
                          preferred_element_type=jnp.float32)
        ur = jax.lax.broadcasted_iota(jnp.int32, (R, R), 0)
        vr = jax.lax.broadcasted_iota(jnp.int32, (R, R), 1)
        lstrict = (vr < ur).astype(jnp.float32)
        ranks_all = jnp.dot(lstrict, onehot, precision=jax.lax.Precision.HIGHEST,
                            preferred_element_type=jnp.float32)
        rank = jnp.sum(ranks_all * onehot, axis=1, keepdims=True)
        offs_r = jnp.sum(offsets * onehot, axis=1, keepdims=True)
        pos = (offs_r + rank).astype(jnp.int32)

        pt = (pos == jax.lax.broadcasted_iota(
            jnp.int32, (R, R), 1)).astype(jnp.float32)
        dn = (((0,), (0,)), ((), ()))
        ts_ref[...] = jnp.round(jax.lax.dot_general(
            pt, t_flat, dn, precision=jax.lax.Precision.HIGHEST,
            preferred_element_type=jnp.float32)).astype(jnp.int32)
        es_ref[...] = jnp.round(jax.lax.dot_general(
            pt, e_flat.astype(jnp.float32), dn,
            precision=jax.lax.Precision.HIGHEST,
            preferred_element_type=jnp.float32)).astype(jnp.int32)
        ws_ref[...] = jax.lax.dot_general(
            pt, w_flat, dn, precision=jax.lax.Precision.HIGHEST,
            preferred_element_type=jnp.float32)

        counts_i = jnp.round(counts).astype(jnp.int32)
        offs_i = jnp.round(offsets).astype(jnp.int32)
        off8 = (offs_i // 8) * 8
        nblk = jnp.where(counts_i > 0,
                         (offs_i + counts_i - off8 + (MB - 1)) // MB, 0)
        start8_ref[...] = off8
        nblk_ref[...] = nblk

    gate = jax.lax.dot_general(x, wg_ref[...], (((1,), (1,)), ((), ())),
                               preferred_element_type=jnp.float32)
    up = jax.lax.dot_general(x, wu_ref[...], (((1,), (1,)), ((), ())),
                             preferred_element_type=jnp.float32)
    h = gate * jax.nn.sigmoid(gate) * up
    contrib = jax.lax.dot_general(h, wd_ref[...], (((1,), (1,)), ((), ())),
                                  preferred_element_type=jnp.float32)

    @pl.when(kf == 0)
    def _init():
        shared_ref[...] = contrib

    @pl.when(kf != 0)
    def _acc():
        shared_ref[...] += contrib


def _grouped_kernel(start8_ref, nblk_ref,
                    x_ref, eg_ref, eu_ref, ed_ref, ts_ref, es_ref,
                    ys_ref, xs_ref):
    e = pl.program_id(0)
    kf = pl.program_id(1)
    j = pl.program_id(2)

    @pl.when((e == 0) & (kf == 0) & (j == 0))
    def _zero():
        ys_ref[...] = jnp.zeros((R, H), jnp.float32)

    @pl.when(j < nblk_ref[0, e])
    def _active():
        lo = start8_ref[0, e] + j * MB
        start = pl.multiple_of(jnp.minimum(lo, R - MB), 8)
        js = pl.multiple_of(j * MB, MB)

        @pl.when(kf == 0)
        def _gather():
            row_t = ts_ref[pl.ds(start, MB), :]
            row_e = es_ref[pl.ds(start, MB), :]
            srow = start + jax.lax.broadcasted_iota(jnp.int32, (MB, 1), 0)
            tok = jax.lax.broadcasted_iota(jnp.int32, (MB, T), 1)
            g = ((row_t == tok) & (row_e == e)
                 & (srow >= lo) & (srow < lo + MB)).astype(jnp.float32)
            xs_ref[pl.ds(js, MB), :] = jnp.dot(
                g, x_ref[...], preferred_element_type=jnp.float32)

        xb = xs_ref[pl.ds(js, MB), :]
        gate = jax.lax.dot_general(xb, eg_ref[0], (((1,), (1,)), ((), ())),
                                   preferred_element_type=jnp.float32)
        up = jax.lax.dot_general(xb, eu_ref[0], (((1,), (1,)), ((), ())),
                                 preferred_element_type=jnp.float32)
        h = gate * jax.nn.sigmoid(gate) * up
        contrib = jax.lax.dot_general(h, ed_ref[0], (((1,), (1,)), ((), ())),
                                      preferred_element_type=jnp.float32)
        ys_ref[pl.ds(start, MB), :] += contrib


def _combine_kernel(ys_ref, shared_ref, ts_ref, ws_ref, out_ref):
    ti = jax.lax.broadcasted_iota(jnp.int32, (R, T), 1)
    ct = jnp.where(ts_ref[...] == ti, ws_ref[...], jnp.float32(0.0))
    out_ref[...] = shared_ref[...] + jax.lax.dot_general(
        ct, ys_ref[...], (((0,), (0,)), ((), ())),
        preferred_element_type=jnp.float32)


@jax.jit
def kernel(x, router_weight, w_gate, w_up, w_down,
           expert_gate, expert_up, expert_down):
    shared_out, ts_col, es_col, ws_col, start8, nblk = pl.pallas_call(
        _router_shared_kernel,
        grid=(KF,),
        in_specs=[
            pl.BlockSpec((T, H), lambda kf: (0, 0)),
            pl.BlockSpec((E, H), lambda kf: (0, 0)),
            pl.BlockSpec((FC, H), lambda kf: (kf, 0)),
            pl.BlockSpec((FC, H), lambda kf: (kf, 0)),
            pl.BlockSpec((H, FC), lambda kf: (0, kf)),
        ],
        out_specs=[
            pl.BlockSpec((T, H), lambda kf: (0, 0)),
            pl.BlockSpec((R, 1), lambda kf: (0, 0)),
            pl.BlockSpec((R, 1), lambda kf: (0, 0)),
            pl.BlockSpec((R, 1), lambda kf: (0, 0)),
            pl.BlockSpec((1, E), lambda kf: (0, 0)),
            pl.BlockSpec((1, E), lambda kf: (0, 0)),
        ],
        out_shape=[
            jax.ShapeDtypeStruct((T, H), jnp.float32),
            jax.ShapeDtypeStruct((R, 1), jnp.int32),
            jax.ShapeDtypeStruct((R, 1), jnp.int32),
            jax.ShapeDtypeStruct((R, 1), jnp.float32),
            jax.ShapeDtypeStruct((1, E), jnp.int32),
            jax.ShapeDtypeStruct((1, E), jnp.int32),
        ],
    )(x, router_weight, w_gate, w_up, w_down)

    ys = pl.pallas_call(
        _grouped_kernel,
        grid_spec=pltpu.PrefetchScalarGridSpec(
            num_scalar_prefetch=2,
            grid=(E, KF, JMAX),
            in_specs=[
                pl.BlockSpec((T, H), lambda e, kf, j, s8, nb: (0, 0)),
                pl.BlockSpec((1, FC, H), lambda e, kf, j, s8, nb: (e, kf, 0)),
                pl.BlockSpec((1, FC, H), lambda e, kf, j, s8, nb: (e, kf, 0)),
                pl.BlockSpec((1, H, FC), lambda e, kf, j, s8, nb: (e, 0, kf)),
                pl.BlockSpec((R, 1), lambda e, kf, j, s8, nb: (0, 0)),
                pl.BlockSpec((R, 1), lambda e, kf, j, s8, nb: (0, 0)),
            ],
            out_specs=pl.BlockSpec((R, H), lambda e, kf, j, s8, nb: (0, 0)),
            scratch_shapes=[pltpu.VMEM((R, H), jnp.float32)],
        ),
        out_shape=jax.ShapeDtypeStruct((R, H), jnp.float32),
    )(start8, nblk, x, expert_gate, expert_up, expert_down, ts_col, es_col)

    out = pl.pallas_call(
        _combine_kernel,
        in_specs=[
            pl.BlockSpec((R, H), lambda: (0, 0)),
            pl.BlockSpec((T, H), lambda: (0, 0)),
            pl.BlockSpec((R, 1), lambda: (0, 0)),
            pl.BlockSpec((R, 1), lambda: (0, 0)),
        ],
        out_specs=pl.BlockSpec((T, H), lambda: (0, 0)),
        out_shape=jax.ShapeDtypeStruct((T, H), jnp.float32),
    )(ys, shared_out, ts_col, ws_col)
    return out


# speedup vs baseline: 1.6509x; 1.6509x over previous
import jax
import jax.numpy as jnp
from jax.experimental import pallas as pl
from jax.experimental.pallas import tpu as pltpu

H = 1024
FF = 2048
E = 8
TOP_K = 2
T = 512

KF = 4
FC = FF // KF
R = T * TOP_K
MB = 128
JMAX = R // MB

_NEG = -3.0e38


def _router_shared_kernel(x_ref, rw_ref, wg_ref, wu_ref, wd_ref,
                          shared_ref, ts_ref, es_ref, ws_ref,
                          start8_ref, nblk_ref):
    kf = pl.program_id(0)
    x = x_ref[...]

    @pl.when(kf == 0)
    def _router_and_dispatch():
        logits = jax.lax.dot_general(
            x, rw_ref[...], (((1,), (1,)), ((), ())),
            preferred_element_type=jnp.float32)
        ii = jax.lax.broadcasted_iota(jnp.int32, (T, E), 1)
        m1 = jnp.max(logits, axis=1, keepdims=True)
        i1 = jnp.min(jnp.where(logits == m1, ii, E), axis=1, keepdims=True)
        l2 = jnp.where(ii == i1, _NEG, logits)
        m2 = jnp.max(l2, axis=1, keepdims=True)
        i2 = jnp.min(jnp.where(l2 == m2, ii, E), axis=1, keepdims=True)
        w1 = jax.nn.sigmoid(m1 - m2)
        w2 = jax.nn.sigmoid(m2 - m1)

        e_flat = jnp.concatenate([i1, i2], axis=0)
        w_flat = jnp.concatenate([w1, w2], axis=0)
        t_col = jax.lax.broadcasted_iota(jnp.int32, (T, 1), 0)
        t_flat = jnp.concatenate([t_col, t_col], axis=0).astype(jnp.float32)

        onehot = (e_flat == jax.lax.broadcasted_iota(
            jnp.int32, (R, E), 1)).astype(jnp.float32)
        counts = jnp.sum(onehot, axis=0, keepdims=True)
        ue = jax.lax.broadcasted_iota(jnp.int32, (E, E), 0)
        ve = jax.lax.broadcasted_iota(jnp.int32, (E, E), 1)
        ustrict = (ue < ve).astype(jnp.float32)
        offsets = jnp.dot(counts, ustrict, precision=jax.lax.Precision.HIGHEST,
                          preferred_element_type=jnp.float32)
        ur = jax.lax.broadcasted_iota(jnp.int32, (R, R), 0)
        vr = jax.lax.broadcasted_iota(jnp.int32, (R, R), 1)
        lstrict = (vr < ur).astype(jnp.float32)
        ranks_all = jnp.dot(lstrict, onehot, precision=jax.lax.Precision.HIGHEST,
                            preferred_element_type=jnp.float32)
        rank = jnp.sum(ranks_all * onehot, axis=1, keepdims=True)
        offs_r = jnp.sum(offsets * onehot, axis=1, keepdims=True)
        pos = (offs_r + rank).astype(jnp.int32)

        pt = (pos == jax.lax.broadcasted_iota(
            jnp.int32, (R, R), 1)).astype(jnp.float32)
        dn = (((0,), (0,)), ((), ()))
        ts_ref[...] = jnp.round(jax.lax.dot_general(
            pt, t_flat, dn, precision=jax.lax.Precision.HIGHEST,
            preferred_element_type=jnp.float32)).astype(jnp.int32)
        es_ref[...] = jnp.round(jax.lax.dot_general(
            pt, e_flat.astype(jnp.float32), dn,
            precision=jax.lax.Precision.HIGHEST,
            preferred_element_type=jnp.float32)).astype(jnp.int32)
        ws_ref[...] = jax.lax.dot_general(
            pt, w_flat, dn, precision=jax.lax.Precision.HIGHEST,
            preferred_element_type=jnp.float32)

        counts_i = jnp.round(counts).astype(jnp.int32)
        offs_i = jnp.round(offsets).astype(jnp.int32)
        off8 = (offs_i // 8) * 8
        nblk = jnp.where(counts_i > 0,
                         (offs_i + counts_i - off8 + (MB - 1)) // MB, 0)
        start8_ref[...] = off8
        nblk_ref[...] = nblk

    gate = jax.lax.dot_general(x, wg_ref[...], (((1,), (1,)), ((), ())),
                               preferred_element_type=jnp.float32)
    up = jax.lax.dot_general(x, wu_ref[...], (((1,), (1,)), ((), ())),
                             preferred_element_type=jnp.float32)
    h = gate * jax.nn.sigmoid(gate) * up
    contrib = jax.lax.dot_general(h, wd_ref[...], (((1,), (1,)), ((), ())),
                                  preferred_element_type=jnp.float32)

    @pl.when(kf == 0)
    def _init():
        shared_ref[...] = contrib

    @pl.when(kf != 0)
    def _acc():
        shared_ref[...] += contrib


def _grouped_kernel(start8_ref, nblk_ref,
                    x_ref, eg_ref, eu_ref, ed_ref, ts_ref, es_ref,
                    ys_ref, xs_ref):
    e = pl.program_id(0)
    kf = pl.program_id(1)

    @pl.when((e == 0) & (kf == 0))
    def _zero():
        ys_ref[...] = jnp.zeros((R, H), jnp.float32)

    def _block(j, carry):
        lo = start8_ref[0, e] + j * MB
        start = pl.multiple_of(jnp.minimum(lo, R - MB), 8)
        js = pl.multiple_of(j * MB, MB)

        @pl.when(kf == 0)
        def _gather():
            row_t = ts_ref[pl.ds(start, MB), :]
            row_e = es_ref[pl.ds(start, MB), :]
            srow = start + jax.lax.broadcasted_iota(jnp.int32, (MB, 1), 0)
            tok = jax.lax.broadcasted_iota(jnp.int32, (MB, T), 1)
            g = ((row_t == tok) & (row_e == e)
                 & (srow >= lo) & (srow < lo + MB)).astype(jnp.float32)
            xs_ref[pl.ds(js, MB), :] = jnp.dot(
                g, x_ref[...], preferred_element_type=jnp.float32)

        xb = xs_ref[pl.ds(js, MB), :]
        gate = jax.lax.dot_general(xb, eg_ref[0], (((1,), (1,)), ((), ())),
                                   preferred_element_type=jnp.float32)
        up = jax.lax.dot_general(xb, eu_ref[0], (((1,), (1,)), ((), ())),
                                 preferred_element_type=jnp.float32)
        h = gate * jax.nn.sigmoid(gate) * up
        contrib = jax.lax.dot_general(h, ed_ref[0], (((1,), (1,)), ((), ())),
                                      preferred_element_type=jnp.float32)
        ys_ref[pl.ds(start, MB), :] += contrib
        return carry

    jax.lax.fori_loop(0, nblk_ref[0, e], _block, 0)


def _combine_kernel(ys_ref, shared_ref, ts_ref, ws_ref, out_ref):
    ti = jax.lax.broadcasted_iota(jnp.int32, (R, T), 1)
    ct = jnp.where(ts_ref[...] == ti, ws_ref[...], jnp.float32(0.0))
    out_ref[...] = shared_ref[...] + jax.lax.dot_general(
        ct, ys_ref[...], (((0,), (0,)), ((), ())),
        preferred_element_type=jnp.float32)


def _stage_a(x, router_weight, w_gate, w_up, w_down):
    return pl.pallas_call(
        _router_shared_kernel,
        grid=(KF,),
        in_specs=[
            pl.BlockSpec((T, H), lambda kf: (0, 0)),
            pl.BlockSpec((E, H), lambda kf: (0, 0)),
            pl.BlockSpec((FC, H), lambda kf: (kf, 0)),
            pl.BlockSpec((FC, H), lambda kf: (kf, 0)),
            pl.BlockSpec((H, FC), lambda kf: (0, kf)),
        ],
        out_specs=[
            pl.BlockSpec((T, H), lambda kf: (0, 0)),
            pl.BlockSpec((R, 1), lambda kf: (0, 0)),
            pl.BlockSpec((R, 1), lambda kf: (0, 0)),
            pl.BlockSpec((R, 1), lambda kf: (0, 0)),
            pl.BlockSpec((1, E), lambda kf: (0, 0)),
            pl.BlockSpec((1, E), lambda kf: (0, 0)),
        ],
        out_shape=[
            jax.ShapeDtypeStruct((T, H), jnp.float32),
            jax.ShapeDtypeStruct((R, 1), jnp.int32),
            jax.ShapeDtypeStruct((R, 1), jnp.int32),
            jax.ShapeDtypeStruct((R, 1), jnp.float32),
            jax.ShapeDtypeStruct((1, E), jnp.int32),
            jax.ShapeDtypeStruct((1, E), jnp.int32),
        ],
    )(x, router_weight, w_gate, w_up, w_down)


def _stage_b(start8, nblk, x, expert_gate, expert_up, expert_down,
             ts_col, es_col):
    return pl.pallas_call(
        _grouped_kernel,
        grid_spec=pltpu.PrefetchScalarGridSpec(
            num_scalar_prefetch=2,
            grid=(E, KF),
            in_specs=[
                pl.BlockSpec((T, H), lambda e, kf, s8, nb: (0, 0)),
                pl.BlockSpec((1, FC, H), lambda e, kf, s8, nb: (e, kf, 0)),
                pl.BlockSpec((1, FC, H), lambda e, kf, s8, nb: (e, kf, 0)),
                pl.BlockSpec((1, H, FC), lambda e, kf, s8, nb: (e, 0, kf)),
                pl.BlockSpec((R, 1), lambda e, kf, s8, nb: (0, 0)),
                pl.BlockSpec((R, 1), lambda e, kf, s8, nb: (0, 0)),
            ],
            out_specs=pl.BlockSpec((R, H), lambda e, kf, s8, nb: (0, 0)),
            scratch_shapes=[pltpu.VMEM((R, H), jnp.float32)],
        ),
        out_shape=jax.ShapeDtypeStruct((R, H), jnp.float32),
    )(start8, nblk, x, expert_gate, expert_up, expert_down, ts_col, es_col)


def _stage_c(ys, shared_out, ts_col, ws_col):
    return pl.pallas_call(
        _combine_kernel,
        in_specs=[
            pl.BlockSpec((R, H), lambda: (0, 0)),
            pl.BlockSpec((T, H), lambda: (0, 0)),
            pl.BlockSpec((R, 1), lambda: (0, 0)),
            pl.BlockSpec((R, 1), lambda: (0, 0)),
        ],
        out_specs=pl.BlockSpec((T, H), lambda: (0, 0)),
        out_shape=jax.ShapeDtypeStruct((T, H), jnp.float32),
    )(ys, shared_out, ts_col, ws_col)


@jax.jit
def kernel(x, router_weight, w_gate, w_up, w_down,
           expert_gate, expert_up, expert_down):
    shared_out, ts_col, es_col, ws_col, start8, nblk = _stage_a(
        x, router_weight, w_gate, w_up, w_down)
    ys = _stage_b(start8, nblk, x, expert_gate, expert_up, expert_down,
                  ts_col, es_col)
    return _stage_c(ys, shared_out, ts_col, ws_col)


# speedup vs baseline: 1.6712x; 1.0123x over previous
import jax
import jax.numpy as jnp
from jax.experimental import pallas as pl
from jax.experimental.pallas import tpu as pltpu

H = 1024
FF = 2048
E = 8
TOP_K = 2
T = 512

KF = 4
FC = FF // KF
R = T * TOP_K
MB = 128
JMAX = R // MB

_NEG = -3.0e38


def _router_shared_kernel(x_ref, rw_ref, wg_ref, wu_ref, wd_ref,
                          shared_ref, ts_ref, es_ref, ws_ref,
                          start8_ref, nblk_ref):
    kf = pl.program_id(0)
    x = x_ref[...]

    @pl.when(kf == 0)
    def _router_and_dispatch():
        logits = jax.lax.dot_general(
            x, rw_ref[...], (((1,), (1,)), ((), ())),
            preferred_element_type=jnp.float32)
        ii = jax.lax.broadcasted_iota(jnp.int32, (T, E), 1)
        m1 = jnp.max(logits, axis=1, keepdims=True)
        i1 = jnp.min(jnp.where(logits == m1, ii, E), axis=1, keepdims=True)
        l2 = jnp.where(ii == i1, _NEG, logits)
        m2 = jnp.max(l2, axis=1, keepdims=True)
        i2 = jnp.min(jnp.where(l2 == m2, ii, E), axis=1, keepdims=True)
        w1 = jax.nn.sigmoid(m1 - m2)
        w2 = jax.nn.sigmoid(m2 - m1)

        e_flat = jnp.concatenate([i1, i2], axis=0)
        w_flat = jnp.concatenate([w1, w2], axis=0)
        t_col = jax.lax.broadcasted_iota(jnp.int32, (T, 1), 0)
        t_flat = jnp.concatenate([t_col, t_col], axis=0).astype(jnp.float32)

        onehot = (e_flat == jax.lax.broadcasted_iota(
            jnp.int32, (R, E), 1)).astype(jnp.float32)
        counts = jnp.sum(onehot, axis=0, keepdims=True)
        ue = jax.lax.broadcasted_iota(jnp.int32, (E, E), 0)
        ve = jax.lax.broadcasted_iota(jnp.int32, (E, E), 1)
        ustrict = (ue < ve).astype(jnp.float32)
        offsets = jnp.dot(counts, ustrict, precision=jax.lax.Precision.HIGHEST,
                          preferred_element_type=jnp.float32)
        ur = jax.lax.broadcasted_iota(jnp.int32, (R, R), 0)
        vr = jax.lax.broadcasted_iota(jnp.int32, (R, R), 1)
        lstrict = (vr < ur).astype(jnp.float32)
        ranks_all = jnp.dot(lstrict, onehot, precision=jax.lax.Precision.HIGHEST,
                            preferred_element_type=jnp.float32)
        rank = jnp.sum(ranks_all * onehot, axis=1, keepdims=True)
        offs_r = jnp.sum(offsets * onehot, axis=1, keepdims=True)
        pos = (offs_r + rank).astype(jnp.int32)

        pt = (pos == jax.lax.broadcasted_iota(
            jnp.int32, (R, R), 1)).astype(jnp.float32)
        dn = (((0,), (0,)), ((), ()))
        ts_ref[...] = jnp.round(jax.lax.dot_general(
            pt, t_flat, dn, precision=jax.lax.Precision.HIGHEST,
            preferred_element_type=jnp.float32)).astype(jnp.int32)
        es_ref[...] = jnp.round(jax.lax.dot_general(
            pt, e_flat.astype(jnp.float32), dn,
            precision=jax.lax.Precision.HIGHEST,
            preferred_element_type=jnp.float32)).astype(jnp.int32)
        ws_ref[...] = jax.lax.dot_general(
            pt, w_flat, dn, precision=jax.lax.Precision.HIGHEST,
            preferred_element_type=jnp.float32)

        counts_i = jnp.round(counts).astype(jnp.int32)
        offs_i = jnp.round(offsets).astype(jnp.int32)
        off8 = (offs_i // 8) * 8
        nblk = jnp.where(counts_i > 0,
                         (offs_i + counts_i - off8 + (MB - 1)) // MB, 0)
        start8_ref[...] = off8
        nblk_ref[...] = nblk

    gate = jax.lax.dot_general(x, wg_ref[...], (((1,), (1,)), ((), ())),
                               preferred_element_type=jnp.float32)
    up = jax.lax.dot_general(x, wu_ref[...], (((1,), (1,)), ((), ())),
                             preferred_element_type=jnp.float32)
    h = gate * jax.nn.sigmoid(gate) * up
    contrib = jax.lax.dot_general(h, wd_ref[...], (((1,), (1,)), ((), ())),
                                  preferred_element_type=jnp.float32)

    @pl.when(kf == 0)
    def _init():
        shared_ref[...] = contrib

    @pl.when(kf != 0)
    def _acc():
        shared_ref[...] += contrib


def _grouped_kernel(start8_ref, nblk_ref,
                    x_ref, eg_ref, eu_ref, ed_ref, ts_ref, es_ref,
                    ws_ref, shared_ref, out_ref, ys_ref, xs_ref):
    e = pl.program_id(0)
    kf = pl.program_id(1)

    @pl.when((e == 0) & (kf == 0))
    def _zero():
        ys_ref[...] = jnp.zeros((R, H), jnp.float32)

    def _block(j, carry):
        lo = start8_ref[0, e] + j * MB
        start = pl.multiple_of(jnp.minimum(lo, R - MB), 8)
        js = pl.multiple_of(j * MB, MB)

        @pl.when(kf == 0)
        def _gather():
            row_t = ts_ref[pl.ds(start, MB), :]
            row_e = es_ref[pl.ds(start, MB), :]
            srow = start + jax.lax.broadcasted_iota(jnp.int32, (MB, 1), 0)
            tok = jax.lax.broadcasted_iota(jnp.int32, (MB, T), 1)
            g = ((row_t == tok) & (row_e == e)
                 & (srow >= lo) & (srow < lo + MB)).astype(jnp.float32)
            xs_ref[pl.ds(js, MB), :] = jnp.dot(
                g, x_ref[...], preferred_element_type=jnp.float32)

        xb = xs_ref[pl.ds(js, MB), :]
        gate = jax.lax.dot_general(xb, eg_ref[0], (((1,), (1,)), ((), ())),
                                   preferred_element_type=jnp.float32)
        up = jax.lax.dot_general(xb, eu_ref[0], (((1,), (1,)), ((), ())),
                                 preferred_element_type=jnp.float32)
        h = gate * jax.nn.sigmoid(gate) * up
        contrib = jax.lax.dot_general(h, ed_ref[0], (((1,), (1,)), ((), ())),
                                      preferred_element_type=jnp.float32)
        ys_ref[pl.ds(start, MB), :] += contrib
        return carry

    jax.lax.fori_loop(0, nblk_ref[0, e], _block, 0)

    @pl.when((e == E - 1) & (kf == KF - 1))
    def _combine():
        ti = jax.lax.broadcasted_iota(jnp.int32, (R, T), 1)
        ct = jnp.where(ts_ref[...] == ti, ws_ref[...], jnp.float32(0.0))
        out_ref[...] = shared_ref[...] + jax.lax.dot_general(
            ct, ys_ref[...], (((0,), (0,)), ((), ())),
            preferred_element_type=jnp.float32)


def _combine_kernel(ys_ref, shared_ref, ts_ref, ws_ref, out_ref):
    ti = jax.lax.broadcasted_iota(jnp.int32, (R, T), 1)
    ct = jnp.where(ts_ref[...] == ti, ws_ref[...], jnp.float32(0.0))
    out_ref[...] = shared_ref[...] + jax.lax.dot_general(
        ct, ys_ref[...], (((0,), (0,)), ((), ())),
        preferred_element_type=jnp.float32)


def _stage_a(x, router_weight, w_gate, w_up, w_down):
    return pl.pallas_call(
        _router_shared_kernel,
        grid=(KF,),
        in_specs=[
            pl.BlockSpec((T, H), lambda kf: (0, 0)),
            pl.BlockSpec((E, H), lambda kf: (0, 0)),
            pl.BlockSpec((FC, H), lambda kf: (kf, 0)),
            pl.BlockSpec((FC, H), lambda kf: (kf, 0)),
            pl.BlockSpec((H, FC), lambda kf: (0, kf)),
        ],
        out_specs=[
            pl.BlockSpec((T, H), lambda kf: (0, 0)),
            pl.BlockSpec((R, 1), lambda kf: (0, 0)),
            pl.BlockSpec((R, 1), lambda kf: (0, 0)),
            pl.BlockSpec((R, 1), lambda kf: (0, 0)),
            pl.BlockSpec((1, E), lambda kf: (0, 0)),
            pl.BlockSpec((1, E), lambda kf: (0, 0)),
        ],
        out_shape=[
            jax.ShapeDtypeStruct((T, H), jnp.float32),
            jax.ShapeDtypeStruct((R, 1), jnp.int32),
            jax.ShapeDtypeStruct((R, 1), jnp.int32),
            jax.ShapeDtypeStruct((R, 1), jnp.float32),
            jax.ShapeDtypeStruct((1, E), jnp.int32),
            jax.ShapeDtypeStruct((1, E), jnp.int32),
        ],
    )(x, router_weight, w_gate, w_up, w_down)


def _stage_b(start8, nblk, x, expert_gate, expert_up, expert_down,
             ts_col, es_col, ws_col, shared_out):
    return pl.pallas_call(
        _grouped_kernel,
        grid_spec=pltpu.PrefetchScalarGridSpec(
            num_scalar_prefetch=2,
            grid=(E, KF),
            in_specs=[
                pl.BlockSpec((T, H), lambda e, kf, s8, nb: (0, 0)),
                pl.BlockSpec((1, FC, H), lambda e, kf, s8, nb: (e, kf, 0)),
                pl.BlockSpec((1, FC, H), lambda e, kf, s8, nb: (e, kf, 0)),
                pl.BlockSpec((1, H, FC), lambda e, kf, s8, nb: (e, 0, kf)),
                pl.BlockSpec((R, 1), lambda e, kf, s8, nb: (0, 0)),
                pl.BlockSpec((R, 1), lambda e, kf, s8, nb: (0, 0)),
                pl.BlockSpec((R, 1), lambda e, kf, s8, nb: (0, 0)),
                pl.BlockSpec((T, H), lambda e, kf, s8, nb: (0, 0)),
            ],
            out_specs=pl.BlockSpec((T, H), lambda e, kf, s8, nb: (0, 0)),
            scratch_shapes=[pltpu.VMEM((R, H), jnp.float32),
                            pltpu.VMEM((R, H), jnp.float32)],
        ),
        out_shape=jax.ShapeDtypeStruct((T, H), jnp.float32),
    )(start8, nblk, x, expert_gate, expert_up, expert_down, ts_col, es_col,
      ws_col, shared_out)


def _stage_c(ys, shared_out, ts_col, ws_col):
    return pl.pallas_call(
        _combine_kernel,
        in_specs=[
            pl.BlockSpec((R, H), lambda: (0, 0)),
            pl.BlockSpec((T, H), lambda: (0, 0)),
            pl.BlockSpec((R, 1), lambda: (0, 0)),
            pl.BlockSpec((R, 1), lambda: (0, 0)),
        ],
        out_specs=pl.BlockSpec((T, H), lambda: (0, 0)),
        out_shape=jax.ShapeDtypeStruct((T, H), jnp.float32),
    )(ys, shared_out, ts_col, ws_col)


@jax.jit
def kernel(x, router_weight, w_gate, w_up, w_down,
           expert_gate, expert_up, expert_down):
    shared_out, ts_col, es_col, ws_col, start8, nblk = _stage_a(
        x, router_weight, w_gate, w_up, w_down)
    return _stage_b(start8, nblk, x, expert_gate, expert_up, expert_down,
                    ts_col, es_col, ws_col, shared_out)
